# Initial kernel scaffold; baseline (speedup 1.0000x reference)
import jax
import jax.numpy as jnp
from jax.experimental import pallas as pl


def kernel(pos_in_feat, pos_edge_index, pos_edge_weight, neg_in_feat, neg_edge_index, neg_edge_weight, W, b, a, Wb, bb):
    raise NotImplementedError("write your pallas kernel here")


# speedup vs baseline: 22.3212x; 22.3212x over previous
import functools

import jax
import jax.numpy as jnp
from jax import lax
from jax.experimental import pallas as pl
from jax.experimental.pallas import tpu as pltpu
from jax.experimental.pallas import tpu_sc as plsc

_N = 10000
_S = 16
_B = _N // _S
_E = 320000
_DIN = 128
_DOUT = 64

_NBINS = _N * _S
_TILES = 16
_EPT = _E // _TILES
_BPT = _NBINS // _TILES


def _sc_build_adjacency(pos_src, pos_dst, pos_ew, neg_src, neg_dst, neg_ew):
    mesh = plsc.VectorSubcoreMesh(core_axis_name="c", subcore_axis_name="s")

    @functools.partial(
        pl.kernel,
        mesh=mesh,
        out_type=jax.ShapeDtypeStruct((2 * _NBINS,), jnp.float32),
        scratch_types=[
            pltpu.VMEM((_EPT,), jnp.int32),
            pltpu.VMEM((_EPT,), jnp.int32),
            pltpu.VMEM((_EPT,), jnp.float32),
            pltpu.VMEM((_BPT,), jnp.float32),
            pltpu.VMEM_SHARED((_NBINS,), jnp.float32),
        ],
    )
    def k(psrc, pdst, pew, nsrc, ndst, new, out, src_v, idx_v, w_v, buf_v,
          acc):
        cid = lax.axis_index("c")
        sid = lax.axis_index("s")

        zeros16 = jnp.zeros((16,), jnp.float32)

        def zbody(i, carry):
            buf_v[pl.ds(i * 16, 16)] = zeros16
            return carry

        lax.fori_loop(0, _BPT // 16, zbody, 0)
        pltpu.sync_copy(buf_v, acc.at[pl.ds(sid * _BPT, _BPT)])
        plsc.subcore_barrier()

        base = sid * _EPT

        def do_branch(esrc, edst, ew):
            pltpu.sync_copy(esrc.at[pl.ds(base, _EPT)], src_v)
            pltpu.sync_copy(edst.at[pl.ds(base, _EPT)], idx_v)
            pltpu.sync_copy(ew.at[pl.ds(base, _EPT)], w_v)

            def body(i, carry):
                s16 = src_v[pl.ds(i * 16, 16)]
                d16 = idx_v[pl.ds(i * 16, 16)]
                idx_v[pl.ds(i * 16, 16)] = (d16 << 4) | (s16 & 15)
                return carry

            lax.fori_loop(0, _EPT // 16, body, 0)
            pltpu.sync_copy(w_v, acc.at[idx_v], add=True)

        @pl.when(cid == 0)
        def _():
            do_branch(psrc, pdst, pew)

        @pl.when(cid == 1)
        def _():
            do_branch(nsrc, ndst, new)

        plsc.subcore_barrier()
        pltpu.sync_copy(acc.at[pl.ds(sid * _BPT, _BPT)], buf_v)
        pltpu.sync_copy(buf_v, out.at[pl.ds(cid * _NBINS + sid * _BPT, _BPT)])

    return k(pos_src, pos_dst, pos_ew, neg_src, neg_dst, neg_ew)


_CB = 125
_CN = _CB * _S
_STEPS = _B // _CB


def _tc_body1(posx_ref, negx_ref, ap_ref, an_ref, w_ref, b_ref, a_ref,
              ppool_out, anch_out, npool_out):
    Wm = w_ref[...]
    alpha = a_ref[0]
    bias = b_ref[...]

    def branch(x, a3):
        xw = jnp.dot(x, Wm, preferred_element_type=jnp.float32)
        xw3 = xw.reshape(_CB, _S, _DOUT)
        agg = jnp.zeros((_CB, _S, _DOUT), jnp.float32)
        for sl in range(_S):
            agg = agg + a3[:, :, sl][:, :, None] * xw3[:, sl, :][:, None, :]
        h = agg + bias[None, None, :]
        h = jnp.where(h >= 0, h, alpha * h)
        pool = h[:, 0, :]
        for dl in range(1, _S - 1):
            pool = pool + h[:, dl, :]
        return pool * (1.0 / (_S - 1)), h[:, _S - 1, :]

    pp, pa = branch(posx_ref[...], ap_ref[...])
    np_, _ = branch(negx_ref[...], an_ref[...])
    ppool_out[...] = pp[None, :, :]
    anch_out[...] = pa[None, :, :]
    npool_out[...] = np_[None, :, :]


def _tc_body2(ppool_ref, anch_ref, npool_ref, wb_ref, bb_ref, pos_out,
              neg_out):
    def l2norm(x):
        n = jnp.sqrt(jnp.sum(x * x, axis=1, keepdims=True))
        return x / jnp.maximum(n, 1e-12)

    pool_p = l2norm(ppool_ref[...])
    pool_n = l2norm(npool_ref[...])
    anchor = l2norm(anch_ref[...])
    Wb0 = wb_ref[...]
    pa = jnp.dot(pool_p, Wb0, preferred_element_type=jnp.float32)
    na = jnp.dot(pool_n, Wb0, preferred_element_type=jnp.float32)
    bias_b = bb_ref[0]
    pos_out[...] = jnp.sum(pa * anchor, axis=1, keepdims=True) + bias_b
    neg_out[...] = jnp.sum(na * anchor, axis=1, keepdims=True) + bias_b


def _tc_dense(pos_x, neg_x, A_pos, A_neg, W, b, Wb0, a, bb):
    ppool, anch, npool = pl.pallas_call(
        _tc_body1,
        grid=(_STEPS,),
        out_shape=(
            jax.ShapeDtypeStruct((_STEPS, _CB, _DOUT), jnp.float32),
            jax.ShapeDtypeStruct((_STEPS, _CB, _DOUT), jnp.float32),
            jax.ShapeDtypeStruct((_STEPS, _CB, _DOUT), jnp.float32),
        ),
        in_specs=[
            pl.BlockSpec((_CN, _DIN), lambda i: (i, 0)),
            pl.BlockSpec((_CN, _DIN), lambda i: (i, 0)),
            pl.BlockSpec((_CB, _S, _S), lambda i: (i, 0, 0)),
            pl.BlockSpec((_CB, _S, _S), lambda i: (i, 0, 0)),
            pl.BlockSpec((_DIN, _DOUT), lambda i: (0, 0)),
            pl.BlockSpec((_DOUT,), lambda i: (0,)),
            pl.BlockSpec(memory_space=pltpu.SMEM),
        ],
        out_specs=(
            pl.BlockSpec((1, _CB, _DOUT), lambda i: (i, 0, 0)),
            pl.BlockSpec((1, _CB, _DOUT), lambda i: (i, 0, 0)),
            pl.BlockSpec((1, _CB, _DOUT), lambda i: (i, 0, 0)),
        ),
    )(pos_x, neg_x, A_pos, A_neg, W, b, a)
    ppool = ppool.reshape(_B, _DOUT)
    anch = anch.reshape(_B, _DOUT)
    npool = npool.reshape(_B, _DOUT)

    return pl.pallas_call(
        _tc_body2,
        out_shape=(
            jax.ShapeDtypeStruct((_B, 1), jnp.float32),
            jax.ShapeDtypeStruct((_B, 1), jnp.float32),
        ),
        in_specs=[
            pl.BlockSpec(memory_space=pltpu.VMEM),
            pl.BlockSpec(memory_space=pltpu.VMEM),
            pl.BlockSpec(memory_space=pltpu.VMEM),
            pl.BlockSpec(memory_space=pltpu.VMEM),
            pl.BlockSpec(memory_space=pltpu.SMEM),
        ],
        out_specs=(
            pl.BlockSpec(memory_space=pltpu.VMEM),
            pl.BlockSpec(memory_space=pltpu.VMEM),
        ),
    )(ppool, anch, npool, Wb0, bb)


def kernel(pos_in_feat, pos_edge_index, pos_edge_weight, neg_in_feat,
           neg_edge_index, neg_edge_weight, W, b, a, Wb, bb):
    A2 = _sc_build_adjacency(pos_edge_index[0], pos_edge_index[1],
                             pos_edge_weight, neg_edge_index[0],
                             neg_edge_index[1], neg_edge_weight)
    pos_scores, neg_scores = _tc_dense(
        pos_in_feat, neg_in_feat, A2[:_NBINS].reshape(_B, _S, _S),
        A2[_NBINS:].reshape(_B, _S, _S), W, b, Wb[0], a, bb)
    return (pos_scores.reshape(_B), neg_scores.reshape(_B))
